# baseline (speedup 1.0000x reference)
import functools

import jax
import jax.numpy as jnp
from jax.experimental import pallas as pl
from jax.experimental.pallas import tpu as pltpu


def _linear_bf16_kernel(x_ref, w_ref, b_ref, o_ref):
    x = x_ref[...].astype(jnp.bfloat16)
    o_ref[...] = (
        jnp.dot(x, w_ref[...], preferred_element_type=jnp.float32) + b_ref[...]
    ).astype(o_ref.dtype)


@functools.partial(jax.jit, static_argnames=("output_size", "tm"))
def _forward(bow_vec, w_p, b_p, *, output_size, tm):
    B, F = bow_vec.shape
    F_pad, O_pad = w_p.shape
    w_bf = w_p.astype(jnp.bfloat16)

    out = pl.pallas_call(
        _linear_bf16_kernel,
        out_shape=jax.ShapeDtypeStruct((B, O_pad), jnp.float32),
        grid=(B // tm,),
        in_specs=[
            pl.BlockSpec((tm, F_pad), lambda i: (i, 0)),
            pl.BlockSpec((F_pad, O_pad), lambda i: (0, 0)),
            pl.BlockSpec((1, O_pad), lambda i: (0, 0)),
        ],
        out_specs=pl.BlockSpec((tm, O_pad), lambda i: (i, 0)),
        compiler_params=pltpu.CompilerParams(
            dimension_semantics=("parallel",),
            vmem_limit_bytes=32 * 1024 * 1024,
        ),
    )(bow_vec, w_bf, b_p)
    return out[:, :output_size]


def kernel(bow_vec, w_p, b_p):
    return _forward(bow_vec, w_p, b_p, output_size=100, tm=256)
